# baseline (speedup 1.0000x reference)
import functools

import jax
import jax.numpy as jnp
from jax import lax
from jax.experimental import pallas as pl
from jax.experimental.pallas import tpu as pltpu
from jax.experimental.pallas import tpu_sc as plsc

N = 10000
E = 320000
IN_CH = 128
H = 32

NC = 2
NS = 16
NW = NC * NS
CHUNK = 128
KDEPTH = 8
NCHUNK = KDEPTH * (-(-E // (NW * CHUNK * KDEPTH)))
E_PAD = NW * NCHUNK * CHUNK
NPAD = 10240
RPT = NPAD // NS
NP4 = NPAD // 4
PW = 4 * H

F32 = jnp.float32

_MESH = dict(core_axis_name="c", subcore_axis_name="s",
             num_cores=NC, num_subcores=NS)


def _sc_degree(dst_idx):
    @functools.partial(
        pl.kernel,
        out_type=jax.ShapeDtypeStruct((NC, NPAD), F32),
        mesh=plsc.VectorSubcoreMesh(**_MESH),
        compiler_params=pltpu.CompilerParams(use_tc_tiling_on_sc=False),
        scratch_types=[
            pltpu.VMEM((NCHUNK, CHUNK), jnp.int32),
            pltpu.VMEM((CHUNK,), F32),
            pltpu.VMEM_SHARED((NPAD,), F32),
        ],
    )
    def deg_kernel(dst_hbm, z_hbm, out_hbm, dst_v, ones_v, acc_s):
        c = lax.axis_index("c")
        s = lax.axis_index("s")
        pltpu.sync_copy(dst_hbm.at[c, s], dst_v)
        for k in range(CHUNK // 16):
            ones_v[pl.ds(k * 16, 16)] = jnp.ones((16,), F32)
        pltpu.sync_copy(z_hbm, acc_s.at[pl.ds(s * RPT, RPT)])
        plsc.subcore_barrier()

        def body(j, carry):
            pltpu.sync_copy(ones_v, acc_s.at[dst_v.at[j]], add=True)
            return carry

        lax.fori_loop(0, NCHUNK, body, 0)
        plsc.subcore_barrier()
        pltpu.sync_copy(acc_s.at[pl.ds(s * RPT, RPT)],
                        out_hbm.at[c, pl.ds(s * RPT, RPT)])

    return deg_kernel(dst_idx, jnp.zeros((RPT,), F32))


def _sc_aggregate(v_tbl, src_idx, dst_idx):
    @functools.partial(
        pl.kernel,
        out_type=jax.ShapeDtypeStruct((NC, NPAD, H), F32),
        mesh=plsc.VectorSubcoreMesh(**_MESH),
        compiler_params=pltpu.CompilerParams(use_tc_tiling_on_sc=False),
        scratch_types=[
            pltpu.VMEM((NCHUNK, CHUNK), jnp.int32),
            pltpu.VMEM((NCHUNK, CHUNK), jnp.int32),
            [pltpu.VMEM((CHUNK, H), F32)] * KDEPTH,
            pltpu.VMEM_SHARED((NPAD, H), F32),
            pltpu.SemaphoreType.DMA,
            pltpu.SemaphoreType.DMA,
        ],
    )
    def agg_kernel(v_hbm, src_hbm, dst_hbm, z_hbm, out_hbm,
                   src_v, dst_v, rbufs, acc_s, gsem, ssem):
        c = lax.axis_index("c")
        s = lax.axis_index("s")
        pltpu.sync_copy(src_hbm.at[c, s], src_v)
        pltpu.sync_copy(dst_hbm.at[c, s], dst_v)
        pltpu.sync_copy(z_hbm, acc_s.at[pl.ds(s * RPT, RPT)])
        plsc.subcore_barrier()

        def body(g, carry):
            j = g * KDEPTH
            descs = []
            for b in range(KDEPTH):
                descs.append(
                    pltpu.async_copy(v_hbm.at[src_v.at[j + b]], rbufs[b], gsem))
            for d in descs:
                d.wait()
            descs = []
            for b in range(KDEPTH):
                descs.append(
                    pltpu.async_copy(rbufs[b], acc_s.at[dst_v.at[j + b]],
                                     ssem, add=True))
            for d in descs:
                d.wait()
            return carry

        lax.fori_loop(0, NCHUNK // KDEPTH, body, 0)
        plsc.subcore_barrier()
        pltpu.sync_copy(acc_s.at[pl.ds(s * RPT, RPT)],
                        out_hbm.at[c, pl.ds(s * RPT, RPT)])

    return agg_kernel(v_tbl, src_idx, dst_idx, jnp.zeros((RPT, H), F32))


RB = 640
GRID = NP4 // RB

def _packed_dot(a, w_ref, kdim):
    parts = [jnp.dot(a[:, i * kdim:(i + 1) * kdim], w_ref[...],
                     preferred_element_type=F32)
             for i in range(4)]
    return jnp.concatenate(parts, axis=1)


def _tc_first(x4, W1, d4):
    def body(x_ref, w_ref, d_ref, dinv_ref, v_ref):
        deg = d_ref[0] + d_ref[1] + 1.0
        dinv4 = lax.rsqrt(deg)
        dinv = jnp.broadcast_to(dinv4[:, :, None], (RB, 4, H)).reshape(RB, PW)
        u = _packed_dot(x_ref[...], w_ref, IN_CH)
        dinv_ref[...] = dinv
        v_ref[...] = dinv * u

    return pl.pallas_call(
        body,
        grid=(GRID,),
        in_specs=[
            pl.BlockSpec((RB, 4 * IN_CH), lambda i: (i, 0)),
            pl.BlockSpec((IN_CH, H), lambda i: (0, 0)),
            pl.BlockSpec((NC, RB, 4), lambda i: (0, i, 0)),
        ],
        out_specs=[
            pl.BlockSpec((RB, PW), lambda i: (i, 0)),
            pl.BlockSpec((RB, PW), lambda i: (i, 0)),
        ],
        out_shape=[
            jax.ShapeDtypeStruct((NP4, PW), F32),
            jax.ShapeDtypeStruct((NP4, PW), F32),
        ],
    )(x4, W1, d4)


def _tc_mid(wp, v, dinv, b, Wn):
    def body(wp_ref, v_ref, dinv_ref, b_ref, w_ref, out_ref):
        w = wp_ref[0] + wp_ref[1]
        h = jnp.tanh(dinv_ref[...] * (w + v_ref[...]) + b_ref[...])
        u = _packed_dot(h, w_ref, H)
        out_ref[...] = dinv_ref[...] * u

    return pl.pallas_call(
        body,
        grid=(GRID,),
        in_specs=[
            pl.BlockSpec((NC, RB, PW), lambda i: (0, i, 0)),
            pl.BlockSpec((RB, PW), lambda i: (i, 0)),
            pl.BlockSpec((RB, PW), lambda i: (i, 0)),
            pl.BlockSpec((1, PW), lambda i: (0, 0)),
            pl.BlockSpec((H, H), lambda i: (0, 0)),
        ],
        out_specs=pl.BlockSpec((RB, PW), lambda i: (i, 0)),
        out_shape=jax.ShapeDtypeStruct((NP4, PW), F32),
    )(wp, v, dinv, b, Wn)


def _tc_last(wp, v, dinv, b):
    def body(wp_ref, v_ref, dinv_ref, b_ref, out_ref):
        w = wp_ref[0] + wp_ref[1]
        out_ref[...] = dinv_ref[...] * (w + v_ref[...]) + b_ref[...]

    return pl.pallas_call(
        body,
        grid=(GRID,),
        in_specs=[
            pl.BlockSpec((NC, RB, PW), lambda i: (0, i, 0)),
            pl.BlockSpec((RB, PW), lambda i: (i, 0)),
            pl.BlockSpec((RB, PW), lambda i: (i, 0)),
            pl.BlockSpec((1, PW), lambda i: (0, 0)),
        ],
        out_specs=pl.BlockSpec((RB, PW), lambda i: (i, 0)),
        out_shape=jax.ShapeDtypeStruct((NP4, PW), F32),
    )(wp, v, dinv, b)


def kernel(x, edge_index, W1, b1, W2, b2, W3, b3):
    src = edge_index[0].astype(jnp.int32)
    dst = edge_index[1].astype(jnp.int32)
    npad_e = E_PAD - E
    pad_src = (jnp.arange(npad_e, dtype=jnp.int32) * 37) % N
    pad_dst = N + jnp.arange(npad_e, dtype=jnp.int32) % (NPAD - N)
    src_p = jnp.concatenate([src, pad_src]).reshape(NC, NS, NCHUNK, CHUNK)
    dst_p = jnp.concatenate([dst, pad_dst]).reshape(NC, NS, NCHUNK, CHUNK)

    W3p = jnp.pad(W3, ((0, 0), (0, H - W3.shape[1])))
    b1p = jnp.tile(b1, 4).reshape(1, PW)
    b2p = jnp.tile(b2, 4).reshape(1, PW)
    b3p = jnp.tile(jnp.pad(b3, (0, H - b3.shape[0])), 4).reshape(1, PW)
    x4 = jnp.pad(x, ((0, NPAD - N), (0, 0))).reshape(NP4, 4 * IN_CH)

    degp = _sc_degree(dst_p)
    d4 = degp.reshape(NC, NP4, 4)

    dinvp, v1p = _tc_first(x4, W1, d4)
    w1 = _sc_aggregate(v1p.reshape(NPAD, H), src_p, dst_p)
    v2p = _tc_mid(w1.reshape(NC, NP4, PW), v1p, dinvp, b1p, W2)
    w2 = _sc_aggregate(v2p.reshape(NPAD, H), src_p, dst_p)
    v3p = _tc_mid(w2.reshape(NC, NP4, PW), v2p, dinvp, b2p, W3p)
    w3 = _sc_aggregate(v3p.reshape(NPAD, H), src_p, dst_p)
    outp = _tc_last(w3.reshape(NC, NP4, PW), v3p, dinvp, b3p)
    return outp.reshape(NPAD, H)[:N, :3]


# speedup vs baseline: 1.2109x; 1.2109x over previous
import functools

import jax
import jax.numpy as jnp
from jax import lax
from jax.experimental import pallas as pl
from jax.experimental.pallas import tpu as pltpu
from jax.experimental.pallas import tpu_sc as plsc

N = 10000
E = 320000
IN_CH = 128
H = 32

NC = 2
NS = 16
NW = NC * NS
CHUNK = 128
KDEPTH = 8
NCHUNK = KDEPTH * (-(-E // (NW * CHUNK * KDEPTH)))
E_PAD = NW * NCHUNK * CHUNK
NPAD = 10240
RPT = NPAD // NS
NP4 = NPAD // 4
PW = 4 * H

F32 = jnp.float32

_MESH = dict(core_axis_name="c", subcore_axis_name="s",
             num_cores=NC, num_subcores=NS)


def _sc_degree(dst_idx):
    @functools.partial(
        pl.kernel,
        out_type=jax.ShapeDtypeStruct((NC, NPAD), F32),
        mesh=plsc.VectorSubcoreMesh(**_MESH),
        compiler_params=pltpu.CompilerParams(use_tc_tiling_on_sc=False),
        scratch_types=[
            pltpu.VMEM((NCHUNK, CHUNK), jnp.int32),
            pltpu.VMEM((CHUNK,), F32),
            pltpu.VMEM_SHARED((NPAD,), F32),
        ],
    )
    def deg_kernel(dst_hbm, z_hbm, out_hbm, dst_v, ones_v, acc_s):
        c = lax.axis_index("c")
        s = lax.axis_index("s")
        pltpu.sync_copy(dst_hbm.at[c, s], dst_v)
        for k in range(CHUNK // 16):
            ones_v[pl.ds(k * 16, 16)] = jnp.ones((16,), F32)
        pltpu.sync_copy(z_hbm, acc_s.at[pl.ds(s * RPT, RPT)])
        plsc.subcore_barrier()

        def body(j, carry):
            pltpu.sync_copy(ones_v, acc_s.at[dst_v.at[j]], add=True)
            return carry

        lax.fori_loop(0, NCHUNK, body, 0)
        plsc.subcore_barrier()
        pltpu.sync_copy(acc_s.at[pl.ds(s * RPT, RPT)],
                        out_hbm.at[c, pl.ds(s * RPT, RPT)])

    return deg_kernel(dst_idx, jnp.zeros((RPT,), F32))


def _sc_aggregate(v_tbl, src_idx, dst_idx):
    @functools.partial(
        pl.kernel,
        out_type=jax.ShapeDtypeStruct((NC, NPAD, H), F32),
        mesh=plsc.VectorSubcoreMesh(**_MESH),
        compiler_params=pltpu.CompilerParams(use_tc_tiling_on_sc=False),
        scratch_types=[
            pltpu.VMEM((NCHUNK, CHUNK), jnp.int32),
            pltpu.VMEM((NCHUNK, CHUNK), jnp.int32),
            [[pltpu.VMEM((CHUNK, H), F32)] * KDEPTH for _ in range(2)],
            pltpu.VMEM_SHARED((NPAD, H), F32),
            pltpu.SemaphoreType.DMA,
            pltpu.SemaphoreType.DMA,
        ],
    )
    def agg_kernel(v_hbm, src_hbm, dst_hbm, z_hbm, out_hbm,
                   src_v, dst_v, rsets, acc_s, gsem, ssem):
        c = lax.axis_index("c")
        s = lax.axis_index("s")
        pltpu.sync_copy(src_hbm.at[c, s], src_v)
        pltpu.sync_copy(dst_hbm.at[c, s], dst_v)
        pltpu.sync_copy(z_hbm, acc_s.at[pl.ds(s * RPT, RPT)])
        plsc.subcore_barrier()

        ngroups = NCHUNK // KDEPTH

        def start_gathers(k, bufs):
            for b in range(KDEPTH):
                pltpu.async_copy(v_hbm.at[src_v.at[k * KDEPTH + b]], bufs[b],
                                 gsem)

        def wait_gathers(k, bufs):
            for b in range(KDEPTH):
                pltpu.make_async_copy(v_hbm.at[src_v.at[k * KDEPTH + b]],
                                      bufs[b], gsem).wait()

        def start_scatters(k, bufs):
            for b in range(KDEPTH):
                pltpu.async_copy(bufs[b], acc_s.at[dst_v.at[k * KDEPTH + b]],
                                 ssem, add=True)

        def wait_scatters(k, bufs):
            for b in range(KDEPTH):
                pltpu.make_async_copy(bufs[b],
                                      acc_s.at[dst_v.at[k * KDEPTH + b]],
                                      ssem).wait()

        start_gathers(0, rsets[0])
        start_gathers(1, rsets[1])
        wait_gathers(0, rsets[0])
        start_scatters(0, rsets[0])

        def body(t, carry):
            for p in range(2):
                k = 2 * t + 1 + p
                parity = (1 + p) % 2
                sk = rsets[parity]
                so = rsets[1 - parity]
                wait_scatters(k - 1, so)
                start_gathers(k + 1, so)
                wait_gathers(k, sk)
                start_scatters(k, sk)
            return carry

        lax.fori_loop(0, (ngroups - 2) // 2, body, 0)
        k = ngroups - 1
        wait_scatters(k - 1, rsets[(k + 1) % 2])
        wait_gathers(k, rsets[k % 2])
        start_scatters(k, rsets[k % 2])
        wait_scatters(k, rsets[k % 2])
        plsc.subcore_barrier()
        pltpu.sync_copy(acc_s.at[pl.ds(s * RPT, RPT)],
                        out_hbm.at[c, pl.ds(s * RPT, RPT)])

    return agg_kernel(v_tbl, src_idx, dst_idx, jnp.zeros((RPT, H), F32))


RB = 640
GRID = NP4 // RB

def _packed_dot(a, w_ref, kdim):
    parts = [jnp.dot(a[:, i * kdim:(i + 1) * kdim], w_ref[...],
                     preferred_element_type=F32)
             for i in range(4)]
    return jnp.concatenate(parts, axis=1)


def _tc_first(x4, W1, d4):
    def body(x_ref, w_ref, d_ref, dinv_ref, v_ref):
        deg = d_ref[0] + d_ref[1] + 1.0
        dinv4 = lax.rsqrt(deg)
        dinv = jnp.broadcast_to(dinv4[:, :, None], (RB, 4, H)).reshape(RB, PW)
        u = _packed_dot(x_ref[...], w_ref, IN_CH)
        dinv_ref[...] = dinv
        v_ref[...] = dinv * u

    return pl.pallas_call(
        body,
        grid=(GRID,),
        in_specs=[
            pl.BlockSpec((RB, 4 * IN_CH), lambda i: (i, 0)),
            pl.BlockSpec((IN_CH, H), lambda i: (0, 0)),
            pl.BlockSpec((NC, RB, 4), lambda i: (0, i, 0)),
        ],
        out_specs=[
            pl.BlockSpec((RB, PW), lambda i: (i, 0)),
            pl.BlockSpec((RB, PW), lambda i: (i, 0)),
        ],
        out_shape=[
            jax.ShapeDtypeStruct((NP4, PW), F32),
            jax.ShapeDtypeStruct((NP4, PW), F32),
        ],
    )(x4, W1, d4)


def _tc_mid(wp, v, dinv, b, Wn):
    def body(wp_ref, v_ref, dinv_ref, b_ref, w_ref, out_ref):
        w = wp_ref[0] + wp_ref[1]
        h = jnp.tanh(dinv_ref[...] * (w + v_ref[...]) + b_ref[...])
        u = _packed_dot(h, w_ref, H)
        out_ref[...] = dinv_ref[...] * u

    return pl.pallas_call(
        body,
        grid=(GRID,),
        in_specs=[
            pl.BlockSpec((NC, RB, PW), lambda i: (0, i, 0)),
            pl.BlockSpec((RB, PW), lambda i: (i, 0)),
            pl.BlockSpec((RB, PW), lambda i: (i, 0)),
            pl.BlockSpec((1, PW), lambda i: (0, 0)),
            pl.BlockSpec((H, H), lambda i: (0, 0)),
        ],
        out_specs=pl.BlockSpec((RB, PW), lambda i: (i, 0)),
        out_shape=jax.ShapeDtypeStruct((NP4, PW), F32),
    )(wp, v, dinv, b, Wn)


def _tc_last(wp, v, dinv, b):
    def body(wp_ref, v_ref, dinv_ref, b_ref, out_ref):
        w = wp_ref[0] + wp_ref[1]
        out_ref[...] = dinv_ref[...] * (w + v_ref[...]) + b_ref[...]

    return pl.pallas_call(
        body,
        grid=(GRID,),
        in_specs=[
            pl.BlockSpec((NC, RB, PW), lambda i: (0, i, 0)),
            pl.BlockSpec((RB, PW), lambda i: (i, 0)),
            pl.BlockSpec((RB, PW), lambda i: (i, 0)),
            pl.BlockSpec((1, PW), lambda i: (0, 0)),
        ],
        out_specs=pl.BlockSpec((RB, PW), lambda i: (i, 0)),
        out_shape=jax.ShapeDtypeStruct((NP4, PW), F32),
    )(wp, v, dinv, b)


def kernel(x, edge_index, W1, b1, W2, b2, W3, b3):
    src = edge_index[0].astype(jnp.int32)
    dst = edge_index[1].astype(jnp.int32)
    npad_e = E_PAD - E
    pad_src = (jnp.arange(npad_e, dtype=jnp.int32) * 37) % N
    pad_dst = N + jnp.arange(npad_e, dtype=jnp.int32) % (NPAD - N)
    src_p = jnp.concatenate([src, pad_src]).reshape(NC, NS, NCHUNK, CHUNK)
    dst_p = jnp.concatenate([dst, pad_dst]).reshape(NC, NS, NCHUNK, CHUNK)

    W3p = jnp.pad(W3, ((0, 0), (0, H - W3.shape[1])))
    b1p = jnp.tile(b1, 4).reshape(1, PW)
    b2p = jnp.tile(b2, 4).reshape(1, PW)
    b3p = jnp.tile(jnp.pad(b3, (0, H - b3.shape[0])), 4).reshape(1, PW)
    x4 = jnp.pad(x, ((0, NPAD - N), (0, 0))).reshape(NP4, 4 * IN_CH)

    degp = _sc_degree(dst_p)
    d4 = degp.reshape(NC, NP4, 4)

    dinvp, v1p = _tc_first(x4, W1, d4)
    w1 = _sc_aggregate(v1p.reshape(NPAD, H), src_p, dst_p)
    v2p = _tc_mid(w1.reshape(NC, NP4, PW), v1p, dinvp, b1p, W2)
    w2 = _sc_aggregate(v2p.reshape(NPAD, H), src_p, dst_p)
    v3p = _tc_mid(w2.reshape(NC, NP4, PW), v2p, dinvp, b2p, W3p)
    w3 = _sc_aggregate(v3p.reshape(NPAD, H), src_p, dst_p)
    outp = _tc_last(w3.reshape(NC, NP4, PW), v3p, dinvp, b3p)
    return outp.reshape(NPAD, H)[:N, :3]


# speedup vs baseline: 1.2860x; 1.0620x over previous
import functools

import jax
import jax.numpy as jnp
from jax import lax
from jax.experimental import pallas as pl
from jax.experimental.pallas import tpu as pltpu
from jax.experimental.pallas import tpu_sc as plsc

N = 10000
E = 320000
IN_CH = 128
H = 32

NC = 2
NS = 16
NW = NC * NS
CHUNK = 128
KDEPTH = 8
NCHUNK = 80
NCH_ALL = NW * NCHUNK
NPAD = 10240
RPT = NPAD // NS
NP4 = NPAD // 4
PW = 4 * H

F32 = jnp.float32

_MESH = dict(core_axis_name="c", subcore_axis_name="s",
             num_cores=NC, num_subcores=NS)


def _sc_degree(dst_idx):
    @functools.partial(
        pl.kernel,
        out_type=jax.ShapeDtypeStruct((NC, NPAD), F32),
        mesh=plsc.VectorSubcoreMesh(**_MESH),
        compiler_params=pltpu.CompilerParams(use_tc_tiling_on_sc=False),
        scratch_types=[
            pltpu.VMEM((NCHUNK, CHUNK), jnp.int32),
            pltpu.VMEM((CHUNK,), F32),
            pltpu.VMEM_SHARED((NPAD,), F32),
        ],
    )
    def deg_kernel(dst_hbm, z_hbm, out_hbm, dst_v, ones_v, acc_s):
        c = lax.axis_index("c")
        s = lax.axis_index("s")
        pltpu.sync_copy(dst_hbm.at[c, s], dst_v)
        for k in range(CHUNK // 16):
            ones_v[pl.ds(k * 16, 16)] = jnp.ones((16,), F32)
        pltpu.sync_copy(z_hbm, acc_s.at[pl.ds(s * RPT, RPT)])
        plsc.subcore_barrier()

        def body(j, carry):
            pltpu.sync_copy(ones_v, acc_s.at[dst_v.at[j]], add=True)
            return carry

        lax.fori_loop(0, NCHUNK, body, 0)
        plsc.subcore_barrier()
        pltpu.sync_copy(acc_s.at[pl.ds(s * RPT, RPT)],
                        out_hbm.at[c, pl.ds(s * RPT, RPT)])

    return deg_kernel(dst_idx, jnp.zeros((RPT,), F32))


def _sc_aggregate(v_tbl, src_idx, dst_idx):
    @functools.partial(
        pl.kernel,
        out_type=jax.ShapeDtypeStruct((NC, NPAD, H), F32),
        mesh=plsc.VectorSubcoreMesh(**_MESH),
        compiler_params=pltpu.CompilerParams(use_tc_tiling_on_sc=False),
        scratch_types=[
            pltpu.VMEM((NCHUNK, CHUNK), jnp.int32),
            pltpu.VMEM((NCHUNK, CHUNK), jnp.int32),
            [[pltpu.VMEM((CHUNK, H), F32)] * KDEPTH for _ in range(2)],
            pltpu.VMEM_SHARED((NPAD, H), F32),
            pltpu.SemaphoreType.DMA,
            pltpu.SemaphoreType.DMA,
        ],
    )
    def agg_kernel(v_hbm, src_hbm, dst_hbm, z_hbm, out_hbm,
                   src_v, dst_v, rsets, acc_s, gsem, ssem):
        c = lax.axis_index("c")
        s = lax.axis_index("s")
        pltpu.sync_copy(src_hbm.at[c, s], src_v)
        pltpu.sync_copy(dst_hbm.at[c, s], dst_v)
        pltpu.sync_copy(z_hbm, acc_s.at[pl.ds(s * RPT, RPT)])
        plsc.subcore_barrier()

        ngroups = NCHUNK // KDEPTH

        def start_gathers(k, bufs):
            for b in range(KDEPTH):
                pltpu.async_copy(v_hbm.at[src_v.at[k * KDEPTH + b]], bufs[b],
                                 gsem)

        def wait_gathers(k, bufs):
            for b in range(KDEPTH):
                pltpu.make_async_copy(v_hbm.at[src_v.at[k * KDEPTH + b]],
                                      bufs[b], gsem).wait()

        def start_scatters(k, bufs):
            for b in range(KDEPTH):
                pltpu.async_copy(bufs[b], acc_s.at[dst_v.at[k * KDEPTH + b]],
                                 ssem, add=True)

        def wait_scatters(k, bufs):
            for b in range(KDEPTH):
                pltpu.make_async_copy(bufs[b],
                                      acc_s.at[dst_v.at[k * KDEPTH + b]],
                                      ssem).wait()

        start_gathers(0, rsets[0])
        start_gathers(1, rsets[1])
        wait_gathers(0, rsets[0])
        start_scatters(0, rsets[0])

        def body(t, carry):
            for p in range(2):
                k = 2 * t + 1 + p
                parity = (1 + p) % 2
                sk = rsets[parity]
                so = rsets[1 - parity]
                wait_scatters(k - 1, so)
                start_gathers(k + 1, so)
                wait_gathers(k, sk)
                start_scatters(k, sk)
            return carry

        lax.fori_loop(0, (ngroups - 2) // 2, body, 0)
        k = ngroups - 1
        wait_scatters(k - 1, rsets[(k + 1) % 2])
        wait_gathers(k, rsets[k % 2])
        start_scatters(k, rsets[k % 2])
        wait_scatters(k, rsets[k % 2])
        plsc.subcore_barrier()
        pltpu.sync_copy(acc_s.at[pl.ds(s * RPT, RPT)],
                        out_hbm.at[c, pl.ds(s * RPT, RPT)])

    return agg_kernel(v_tbl, src_idx, dst_idx, jnp.zeros((RPT, H), F32))


RB = 640
GRID = NP4 // RB

def _packed_dot(a, w_ref, kdim):
    parts = [jnp.dot(a[:, i * kdim:(i + 1) * kdim], w_ref[...],
                     preferred_element_type=F32)
             for i in range(4)]
    return jnp.concatenate(parts, axis=1)


def _tc_first(x4, W1, d4):
    def body(x_ref, w_ref, d_ref, dinv_ref, v_ref):
        deg = d_ref[0] + d_ref[1] + 1.0
        dinv4 = lax.rsqrt(deg)
        dinv = jnp.broadcast_to(dinv4[:, :, None], (RB, 4, H)).reshape(RB, PW)
        u = _packed_dot(x_ref[...], w_ref, IN_CH)
        dinv_ref[...] = dinv
        v_ref[...] = dinv * u

    return pl.pallas_call(
        body,
        grid=(GRID,),
        in_specs=[
            pl.BlockSpec((RB, 4 * IN_CH), lambda i: (i, 0)),
            pl.BlockSpec((IN_CH, H), lambda i: (0, 0)),
            pl.BlockSpec((NC, RB, 4), lambda i: (0, i, 0)),
        ],
        out_specs=[
            pl.BlockSpec((RB, PW), lambda i: (i, 0)),
            pl.BlockSpec((RB, PW), lambda i: (i, 0)),
        ],
        out_shape=[
            jax.ShapeDtypeStruct((NP4, PW), F32),
            jax.ShapeDtypeStruct((NP4, PW), F32),
        ],
    )(x4, W1, d4)


def _tc_mid(wp, v, dinv, b, Wn):
    def body(wp_ref, v_ref, dinv_ref, b_ref, w_ref, out_ref):
        w = wp_ref[0] + wp_ref[1]
        h = jnp.tanh(dinv_ref[...] * (w + v_ref[...]) + b_ref[...])
        u = _packed_dot(h, w_ref, H)
        out_ref[...] = dinv_ref[...] * u

    return pl.pallas_call(
        body,
        grid=(GRID,),
        in_specs=[
            pl.BlockSpec((NC, RB, PW), lambda i: (0, i, 0)),
            pl.BlockSpec((RB, PW), lambda i: (i, 0)),
            pl.BlockSpec((RB, PW), lambda i: (i, 0)),
            pl.BlockSpec((1, PW), lambda i: (0, 0)),
            pl.BlockSpec((H, H), lambda i: (0, 0)),
        ],
        out_specs=pl.BlockSpec((RB, PW), lambda i: (i, 0)),
        out_shape=jax.ShapeDtypeStruct((NP4, PW), F32),
    )(wp, v, dinv, b, Wn)


def _tc_last(wp, v, dinv, b):
    def body(wp_ref, v_ref, dinv_ref, b_ref, out_ref):
        w = wp_ref[0] + wp_ref[1]
        out_ref[...] = dinv_ref[...] * (w + v_ref[...]) + b_ref[...]

    return pl.pallas_call(
        body,
        grid=(GRID,),
        in_specs=[
            pl.BlockSpec((NC, RB, PW), lambda i: (0, i, 0)),
            pl.BlockSpec((RB, PW), lambda i: (i, 0)),
            pl.BlockSpec((RB, PW), lambda i: (i, 0)),
            pl.BlockSpec((1, PW), lambda i: (0, 0)),
        ],
        out_specs=pl.BlockSpec((RB, PW), lambda i: (i, 0)),
        out_shape=jax.ShapeDtypeStruct((NP4, PW), F32),
    )(wp, v, dinv, b)


def kernel(x, edge_index, W1, b1, W2, b2, W3, b3):
    ei = edge_index.astype(jnp.int32).reshape(2, E // CHUNK, CHUNK)
    npad_e = (NCH_ALL - E // CHUNK) * CHUNK
    pad_src = ((jnp.arange(npad_e, dtype=jnp.int32) * 37) % N
               ).reshape(1, -1, CHUNK)
    pad_dst = (N + jnp.arange(npad_e, dtype=jnp.int32) % (NPAD - N)
               ).reshape(1, -1, CHUNK)
    pad_cat = jnp.concatenate([pad_src, pad_dst], axis=0)
    ei_p = jnp.concatenate([ei, pad_cat], axis=1)
    src_p = ei_p[0].reshape(NC, NS, NCHUNK, CHUNK)
    dst_p = ei_p[1].reshape(NC, NS, NCHUNK, CHUNK)

    W3p = jnp.pad(W3, ((0, 0), (0, H - W3.shape[1])))
    b1p = jnp.tile(b1, 4).reshape(1, PW)
    b2p = jnp.tile(b2, 4).reshape(1, PW)
    b3p = jnp.tile(jnp.pad(b3, (0, H - b3.shape[0])), 4).reshape(1, PW)
    x4 = jnp.pad(x, ((0, NPAD - N), (0, 0))).reshape(NP4, 4 * IN_CH)

    degp = _sc_degree(dst_p)
    d4 = degp.reshape(NC, NP4, 4)

    dinvp, v1p = _tc_first(x4, W1, d4)
    w1 = _sc_aggregate(v1p.reshape(NPAD, H), src_p, dst_p)
    v2p = _tc_mid(w1.reshape(NC, NP4, PW), v1p, dinvp, b1p, W2)
    w2 = _sc_aggregate(v2p.reshape(NPAD, H), src_p, dst_p)
    v3p = _tc_mid(w2.reshape(NC, NP4, PW), v2p, dinvp, b2p, W3p)
    w3 = _sc_aggregate(v3p.reshape(NPAD, H), src_p, dst_p)
    outp = _tc_last(w3.reshape(NC, NP4, PW), v3p, dinvp, b3p)
    return outp.reshape(NPAD, H)[:N, :3]
